# baseline (device time: 28707 ns/iter reference)
import jax
import jax.numpy as jnp
from jax import lax
from jax.experimental import pallas as pl
from jax.experimental.pallas import tpu as pltpu

T = 512
D = 512
F = 1024
E_LOC = 2
NCHUNK = 4


def kernel(x, assign, W1, W2):
    assign2d = assign.reshape(T, 1)

    def body(x_ref, a_ref, w1_hbm, w2_hbm, out_ref,
             xb, w1f, w2f, w1b, w2b, xrecv, arecv, accrem, partner,
             send_sems, recv_sems, ret_send_sems, ret_recv_sems, w_sems):
        my_x = lax.axis_index("x")
        my_y = lax.axis_index("y")
        nbr = (my_x, 1 - my_y)

        barrier_sem = pltpu.get_barrier_semaphore()
        pl.semaphore_signal(barrier_sem, inc=1, device_id=nbr,
                            device_id_type=pl.DeviceIdType.MESH)
        pl.semaphore_wait(barrier_sem, 1)

        xb[:, :] = x_ref[:, :].astype(jnp.bfloat16)
        rdma_x = pltpu.make_async_remote_copy(
            src_ref=xb, dst_ref=xrecv,
            send_sem=send_sems.at[0], recv_sem=recv_sems.at[0],
            device_id=nbr, device_id_type=pl.DeviceIdType.MESH,
        )
        rdma_a = pltpu.make_async_remote_copy(
            src_ref=a_ref, dst_ref=arecv,
            send_sem=send_sems.at[1], recv_sem=recv_sems.at[1],
            device_id=nbr, device_id_type=pl.DeviceIdType.MESH,
        )
        rdma_x.start()
        rdma_a.start()

        wdmas = []
        for k in range(E_LOC):
            d1 = pltpu.make_async_copy(w1_hbm.at[k], w1f.at[k], w_sems.at[2 * k])
            d2 = pltpu.make_async_copy(w2_hbm.at[k], w2f.at[k], w_sems.at[2 * k + 1])
            d1.start()
            d2.start()
            wdmas.append((d1, d2))
        for k in range(E_LOC):
            d1, d2 = wdmas[k]
            d1.wait()
            w1b[k] = w1f[k].astype(jnp.bfloat16)
            d2.wait()
            w2b[k] = w2f[k].astype(jnp.bfloat16)

        def moe(tokens, assigns):
            acc = jnp.zeros((tokens.shape[0], D), jnp.float32)
            for k in range(E_LOC):
                e = E_LOC * my_y + k
                h = jnp.maximum(
                    jnp.dot(tokens, w1b[k], preferred_element_type=jnp.float32),
                    0.0).astype(jnp.bfloat16)
                y = jnp.dot(h, w2b[k], preferred_element_type=jnp.float32)
                acc = acc + jnp.where(assigns == e, y, 0.0)
            return acc

        acc_local = moe(xb[:, :], a_ref[:, :])

        rdma_x.wait()
        rdma_a.wait()

        CH = T // NCHUNK
        rets = []
        for c in range(NCHUNK):
            rows = slice(c * CH, (c + 1) * CH)
            accrem[rows, :] = moe(xrecv[rows, :], arecv[rows, :]).astype(jnp.bfloat16)
            r = pltpu.make_async_remote_copy(
                src_ref=accrem.at[rows],
                dst_ref=partner.at[rows],
                send_sem=ret_send_sems.at[c], recv_sem=ret_recv_sems.at[c],
                device_id=nbr, device_id_type=pl.DeviceIdType.MESH,
            )
            r.start()
            rets.append(r)

        for c, r in enumerate(rets):
            rows = slice(c * CH, (c + 1) * CH)
            r.wait_recv()
            out_ref[rows, :] = acc_local[rows, :] + partner[rows, :].astype(jnp.float32)
        for r in rets:
            r.wait_send()

    return pl.pallas_call(
        body,
        out_shape=jax.ShapeDtypeStruct((T, D), jnp.float32),
        in_specs=[
            pl.BlockSpec(memory_space=pltpu.VMEM),
            pl.BlockSpec(memory_space=pltpu.VMEM),
            pl.BlockSpec(memory_space=pltpu.HBM),
            pl.BlockSpec(memory_space=pltpu.HBM),
        ],
        out_specs=pl.BlockSpec(memory_space=pltpu.VMEM),
        scratch_shapes=[
            pltpu.VMEM((T, D), jnp.bfloat16),
            pltpu.VMEM((E_LOC, D, F), jnp.float32),
            pltpu.VMEM((E_LOC, F, D), jnp.float32),
            pltpu.VMEM((E_LOC, D, F), jnp.bfloat16),
            pltpu.VMEM((E_LOC, F, D), jnp.bfloat16),
            pltpu.VMEM((T, D), jnp.bfloat16),
            pltpu.VMEM((T, 1), jnp.int32),
            pltpu.VMEM((T, D), jnp.bfloat16),
            pltpu.VMEM((T, D), jnp.bfloat16),
            pltpu.SemaphoreType.DMA((2,)),
            pltpu.SemaphoreType.DMA((2,)),
            pltpu.SemaphoreType.DMA((NCHUNK,)),
            pltpu.SemaphoreType.DMA((NCHUNK,)),
            pltpu.SemaphoreType.DMA((2 * E_LOC,)),
        ],
        compiler_params=pltpu.CompilerParams(collective_id=0),
    )(x, assign2d, W1, W2)


# device time: 26208 ns/iter; 1.0954x vs baseline; 1.0954x over previous
import jax
import jax.numpy as jnp
from jax import lax
from jax.experimental import pallas as pl
from jax.experimental.pallas import tpu as pltpu

T = 512
D = 512
F = 1024
E_LOC = 2
HH = T // 2

S_X, S_A, S_PLOC, S_PREM_S, S_PREM_D = range(5)


def kernel(x, assign, W1, W2):
    assign2d = assign.reshape(T, 1)
    xb_in = x.astype(jnp.bfloat16)
    w1b_in = W1.astype(jnp.bfloat16)
    w2b_in = W2.astype(jnp.bfloat16)

    def body(x_ref, a_ref, w1_ref, w2_ref, out_ref,
             xrecv, arecv, ploc_mine, ploc_recv, prem_send,
             prem_recv_s, prem_recv_d, send_sems, recv_sems):
        my_x = lax.axis_index("x")
        my_y = lax.axis_index("y")
        x_nbr = (1 - my_x, my_y)
        y_nbr = (my_x, 1 - my_y)
        diag = (1 - my_x, 1 - my_y)

        my_half = pl.ds(my_x * HH, HH)

        barrier_sem = pltpu.get_barrier_semaphore()
        for nbr in (x_nbr, y_nbr, diag):
            pl.semaphore_signal(barrier_sem, inc=1, device_id=nbr,
                                device_id_type=pl.DeviceIdType.MESH)
        pl.semaphore_wait(barrier_sem, 3)

        rdma_x = pltpu.make_async_remote_copy(
            src_ref=x_ref.at[my_half], dst_ref=xrecv,
            send_sem=send_sems.at[S_X], recv_sem=recv_sems.at[S_X],
            device_id=y_nbr, device_id_type=pl.DeviceIdType.MESH,
        )
        rdma_a = pltpu.make_async_remote_copy(
            src_ref=a_ref, dst_ref=arecv,
            send_sem=send_sems.at[S_A], recv_sem=recv_sems.at[S_A],
            device_id=y_nbr, device_id_type=pl.DeviceIdType.MESH,
        )
        rdma_x.start()
        rdma_a.start()

        def moe(tokens, assigns):
            acc = jnp.zeros((tokens.shape[0], D), jnp.float32)
            for k in range(E_LOC):
                e = E_LOC * my_y + k
                h = jnp.maximum(
                    jnp.dot(tokens, w1_ref[k], preferred_element_type=jnp.float32),
                    0.0).astype(jnp.bfloat16)
                y = jnp.dot(h, w2_ref[k], preferred_element_type=jnp.float32)
                acc = acc + jnp.where(assigns == e, y, 0.0)
            return acc

        ploc_mine[:, :] = moe(x_ref[my_half, :], a_ref[my_half, :]).astype(jnp.bfloat16)
        rdma_ploc = pltpu.make_async_remote_copy(
            src_ref=ploc_mine, dst_ref=ploc_recv,
            send_sem=send_sems.at[S_PLOC], recv_sem=recv_sems.at[S_PLOC],
            device_id=x_nbr, device_id_type=pl.DeviceIdType.MESH,
        )
        rdma_ploc.start()

        rdma_x.wait_recv()
        rdma_a.wait_recv()
        prem_send[:, :] = moe(xrecv[:, :], arecv[my_half, :]).astype(jnp.bfloat16)
        rdma_prem_s = pltpu.make_async_remote_copy(
            src_ref=prem_send, dst_ref=prem_recv_s,
            send_sem=send_sems.at[S_PREM_S], recv_sem=recv_sems.at[S_PREM_S],
            device_id=y_nbr, device_id_type=pl.DeviceIdType.MESH,
        )
        rdma_prem_d = pltpu.make_async_remote_copy(
            src_ref=prem_send, dst_ref=prem_recv_d,
            send_sem=send_sems.at[S_PREM_D], recv_sem=recv_sems.at[S_PREM_D],
            device_id=diag, device_id_type=pl.DeviceIdType.MESH,
        )
        rdma_prem_s.start()
        rdma_prem_d.start()

        other_half = pl.ds((1 - my_x) * HH, HH)
        rdma_prem_s.wait_recv()
        out_ref[my_half, :] = (ploc_mine[:, :].astype(jnp.float32)
                               + prem_recv_s[:, :].astype(jnp.float32))
        rdma_ploc.wait_recv()
        rdma_prem_d.wait_recv()
        out_ref[other_half, :] = (ploc_recv[:, :].astype(jnp.float32)
                                  + prem_recv_d[:, :].astype(jnp.float32))

        rdma_x.wait_send()
        rdma_a.wait_send()
        rdma_ploc.wait_send()
        rdma_prem_s.wait_send()
        rdma_prem_d.wait_send()

    return pl.pallas_call(
        body,
        out_shape=jax.ShapeDtypeStruct((T, D), jnp.float32),
        in_specs=[
            pl.BlockSpec(memory_space=pltpu.VMEM),
            pl.BlockSpec(memory_space=pltpu.VMEM),
            pl.BlockSpec(memory_space=pltpu.VMEM),
            pl.BlockSpec(memory_space=pltpu.VMEM),
        ],
        out_specs=pl.BlockSpec(memory_space=pltpu.VMEM),
        scratch_shapes=[
            pltpu.VMEM((HH, D), jnp.bfloat16),
            pltpu.VMEM((T, 1), jnp.int32),
            pltpu.VMEM((HH, D), jnp.bfloat16),
            pltpu.VMEM((HH, D), jnp.bfloat16),
            pltpu.VMEM((HH, D), jnp.bfloat16),
            pltpu.VMEM((HH, D), jnp.bfloat16),
            pltpu.VMEM((HH, D), jnp.bfloat16),
            pltpu.SemaphoreType.DMA((5,)),
            pltpu.SemaphoreType.DMA((5,)),
        ],
        compiler_params=pltpu.CompilerParams(collective_id=0),
    )(xb_in, assign2d, w1b_in, w2b_in)


# device time: 25418 ns/iter; 1.1294x vs baseline; 1.0311x over previous
import jax
import jax.numpy as jnp
from jax import lax
from jax.experimental import pallas as pl
from jax.experimental.pallas import tpu as pltpu

T = 512
D = 512
F = 1024
E_LOC = 2
HH = T // 2
NC = 2
CH = HH // NC

S_X, S_A = 0, 1
S_PLOC = 2
S_PREM_S = S_PLOC + NC
S_PREM_D = S_PREM_S + NC
NSEM = S_PREM_D + NC


def kernel(x, assign, W1, W2):
    assign2d = assign.reshape(T, 1)
    w1b_in = W1.astype(jnp.bfloat16)
    w2b_in = W2.astype(jnp.bfloat16)

    def body(x_ref, a_ref, w1_ref, w2_ref, out_ref,
             xb, xrecv, arecv, ploc_mine, ploc_recv, prem_send,
             prem_recv_s, prem_recv_d, send_sems, recv_sems):
        my_x = lax.axis_index("x")
        my_y = lax.axis_index("y")
        x_nbr = (1 - my_x, my_y)
        y_nbr = (my_x, 1 - my_y)
        diag = (1 - my_x, 1 - my_y)

        half0 = my_x * HH

        barrier_sem = pltpu.get_barrier_semaphore()
        for nbr in (x_nbr, y_nbr, diag):
            pl.semaphore_signal(barrier_sem, inc=1, device_id=nbr,
                                device_id_type=pl.DeviceIdType.MESH)
        pl.semaphore_wait(barrier_sem, 3)

        xb[:, :] = x_ref[pl.ds(half0, HH), :].astype(jnp.bfloat16)
        rdma_x = pltpu.make_async_remote_copy(
            src_ref=xb, dst_ref=xrecv,
            send_sem=send_sems.at[S_X], recv_sem=recv_sems.at[S_X],
            device_id=y_nbr, device_id_type=pl.DeviceIdType.MESH,
        )
        rdma_a = pltpu.make_async_remote_copy(
            src_ref=a_ref, dst_ref=arecv,
            send_sem=send_sems.at[S_A], recv_sem=recv_sems.at[S_A],
            device_id=y_nbr, device_id_type=pl.DeviceIdType.MESH,
        )
        rdma_x.start()
        rdma_a.start()

        def moe(tokens, assigns):
            acc = jnp.zeros((tokens.shape[0], D), jnp.float32)
            for k in range(E_LOC):
                e = E_LOC * my_y + k
                h = jnp.maximum(
                    jnp.dot(tokens, w1_ref[k], preferred_element_type=jnp.float32),
                    0.0).astype(jnp.bfloat16)
                y = jnp.dot(h, w2_ref[k], preferred_element_type=jnp.float32)
                acc = acc + jnp.where(assigns == e, y, 0.0)
            return acc

        ploc_rdmas = []
        for c in range(NC):
            rows = slice(c * CH, (c + 1) * CH)
            ploc_mine[rows, :] = moe(
                x_ref[pl.ds(half0 + c * CH, CH), :].astype(jnp.bfloat16),
                a_ref[pl.ds(half0 + c * CH, CH), :],
            ).astype(jnp.bfloat16)
            r = pltpu.make_async_remote_copy(
                src_ref=ploc_mine.at[rows], dst_ref=ploc_recv.at[rows],
                send_sem=send_sems.at[S_PLOC + c], recv_sem=recv_sems.at[S_PLOC + c],
                device_id=x_nbr, device_id_type=pl.DeviceIdType.MESH,
            )
            r.start()
            ploc_rdmas.append(r)

        rdma_x.wait_recv()
        rdma_a.wait_recv()
        prem_rdmas = []
        for c in range(NC):
            rows = slice(c * CH, (c + 1) * CH)
            prem_send[rows, :] = moe(
                xrecv[rows, :],
                arecv[pl.ds(half0 + c * CH, CH), :],
            ).astype(jnp.bfloat16)
            rs = pltpu.make_async_remote_copy(
                src_ref=prem_send.at[rows], dst_ref=prem_recv_s.at[rows],
                send_sem=send_sems.at[S_PREM_S + c],
                recv_sem=recv_sems.at[S_PREM_S + c],
                device_id=y_nbr, device_id_type=pl.DeviceIdType.MESH,
            )
            rd = pltpu.make_async_remote_copy(
                src_ref=prem_send.at[rows], dst_ref=prem_recv_d.at[rows],
                send_sem=send_sems.at[S_PREM_D + c],
                recv_sem=recv_sems.at[S_PREM_D + c],
                device_id=diag, device_id_type=pl.DeviceIdType.MESH,
            )
            rs.start()
            rd.start()
            prem_rdmas.append((rs, rd))

        other0 = (1 - my_x) * HH
        for c in range(NC):
            rows = slice(c * CH, (c + 1) * CH)
            prem_rdmas[c][0].wait_recv()
            out_ref[pl.ds(half0 + c * CH, CH), :] = (
                ploc_mine[rows, :].astype(jnp.float32)
                + prem_recv_s[rows, :].astype(jnp.float32))
        for c in range(NC):
            rows = slice(c * CH, (c + 1) * CH)
            ploc_rdmas[c].wait_recv()
            prem_rdmas[c][1].wait_recv()
            out_ref[pl.ds(other0 + c * CH, CH), :] = (
                ploc_recv[rows, :].astype(jnp.float32)
                + prem_recv_d[rows, :].astype(jnp.float32))

        rdma_x.wait_send()
        rdma_a.wait_send()
        for r in ploc_rdmas:
            r.wait_send()
        for rs, rd in prem_rdmas:
            rs.wait_send()
            rd.wait_send()

    return pl.pallas_call(
        body,
        out_shape=jax.ShapeDtypeStruct((T, D), jnp.float32),
        in_specs=[
            pl.BlockSpec(memory_space=pltpu.VMEM),
            pl.BlockSpec(memory_space=pltpu.VMEM),
            pl.BlockSpec(memory_space=pltpu.VMEM),
            pl.BlockSpec(memory_space=pltpu.VMEM),
        ],
        out_specs=pl.BlockSpec(memory_space=pltpu.VMEM),
        scratch_shapes=[
            pltpu.VMEM((HH, D), jnp.bfloat16),
            pltpu.VMEM((HH, D), jnp.bfloat16),
            pltpu.VMEM((T, 1), jnp.int32),
            pltpu.VMEM((HH, D), jnp.bfloat16),
            pltpu.VMEM((HH, D), jnp.bfloat16),
            pltpu.VMEM((HH, D), jnp.bfloat16),
            pltpu.VMEM((HH, D), jnp.bfloat16),
            pltpu.VMEM((HH, D), jnp.bfloat16),
            pltpu.SemaphoreType.DMA((NSEM,)),
            pltpu.SemaphoreType.DMA((NSEM,)),
        ],
        compiler_params=pltpu.CompilerParams(collective_id=0),
    )(x, assign2d, w1b_in, w2b_in)


# device time: 25238 ns/iter; 1.1375x vs baseline; 1.0071x over previous
import jax
import jax.numpy as jnp
from jax import lax
from jax.experimental import pallas as pl
from jax.experimental.pallas import tpu as pltpu

T = 512
D = 512
F = 1024
E_LOC = 2
HH = T // 2
NC = 4
CH = HH // NC

S_X, S_A = 0, 1
S_PLOC = 2
S_PREM_S = S_PLOC + NC
S_PREM_D = S_PREM_S + NC
NSEM = S_PREM_D + NC


def kernel(x, assign, W1, W2):
    assign2d = assign.reshape(T, 1)
    w1b_in = W1.astype(jnp.bfloat16)
    w2b_in = W2.astype(jnp.bfloat16)

    def body(x_ref, a_ref, w1_ref, w2_ref, out_ref,
             xb, xrecv, arecv, ploc_mine, ploc_recv, prem_send,
             prem_recv_s, prem_recv_d, send_sems, recv_sems):
        my_x = lax.axis_index("x")
        my_y = lax.axis_index("y")
        x_nbr = (1 - my_x, my_y)
        y_nbr = (my_x, 1 - my_y)
        diag = (1 - my_x, 1 - my_y)

        half0 = my_x * HH

        barrier_sem = pltpu.get_barrier_semaphore()
        for nbr in (x_nbr, y_nbr, diag):
            pl.semaphore_signal(barrier_sem, inc=1, device_id=nbr,
                                device_id_type=pl.DeviceIdType.MESH)
        pl.semaphore_wait(barrier_sem, 3)

        xb[:, :] = x_ref[pl.ds(half0, HH), :].astype(jnp.bfloat16)
        rdma_x = pltpu.make_async_remote_copy(
            src_ref=xb, dst_ref=xrecv,
            send_sem=send_sems.at[S_X], recv_sem=recv_sems.at[S_X],
            device_id=y_nbr, device_id_type=pl.DeviceIdType.MESH,
        )
        rdma_a = pltpu.make_async_remote_copy(
            src_ref=a_ref, dst_ref=arecv,
            send_sem=send_sems.at[S_A], recv_sem=recv_sems.at[S_A],
            device_id=y_nbr, device_id_type=pl.DeviceIdType.MESH,
        )
        rdma_x.start()
        rdma_a.start()

        def moe(tokens, assigns):
            acc = jnp.zeros((tokens.shape[0], D), jnp.float32)
            for k in range(E_LOC):
                e = E_LOC * my_y + k
                h = jnp.maximum(
                    jnp.dot(tokens, w1_ref[k], preferred_element_type=jnp.float32),
                    0.0).astype(jnp.bfloat16)
                y = jnp.dot(h, w2_ref[k], preferred_element_type=jnp.float32)
                acc = acc + jnp.where(assigns == e, y, 0.0)
            return acc

        ploc_rdmas = []
        for c in range(NC):
            rows = slice(c * CH, (c + 1) * CH)
            ploc_mine[rows, :] = moe(
                xb[rows, :],
                a_ref[pl.ds(half0 + c * CH, CH), :],
            ).astype(jnp.bfloat16)
            r = pltpu.make_async_remote_copy(
                src_ref=ploc_mine.at[rows], dst_ref=ploc_recv.at[rows],
                send_sem=send_sems.at[S_PLOC + c], recv_sem=recv_sems.at[S_PLOC + c],
                device_id=x_nbr, device_id_type=pl.DeviceIdType.MESH,
            )
            r.start()
            ploc_rdmas.append(r)

        rdma_x.wait_recv()
        rdma_a.wait_recv()
        prem_rdmas = []
        for c in range(NC):
            rows = slice(c * CH, (c + 1) * CH)
            prem_send[rows, :] = moe(
                xrecv[rows, :],
                arecv[pl.ds(half0 + c * CH, CH), :],
            ).astype(jnp.bfloat16)
            rs = pltpu.make_async_remote_copy(
                src_ref=prem_send.at[rows], dst_ref=prem_recv_s.at[rows],
                send_sem=send_sems.at[S_PREM_S + c],
                recv_sem=recv_sems.at[S_PREM_S + c],
                device_id=y_nbr, device_id_type=pl.DeviceIdType.MESH,
            )
            rd = pltpu.make_async_remote_copy(
                src_ref=prem_send.at[rows], dst_ref=prem_recv_d.at[rows],
                send_sem=send_sems.at[S_PREM_D + c],
                recv_sem=recv_sems.at[S_PREM_D + c],
                device_id=diag, device_id_type=pl.DeviceIdType.MESH,
            )
            rs.start()
            rd.start()
            prem_rdmas.append((rs, rd))

        other0 = (1 - my_x) * HH
        for c in range(NC):
            rows = slice(c * CH, (c + 1) * CH)
            prem_rdmas[c][0].wait_recv()
            out_ref[pl.ds(half0 + c * CH, CH), :] = (
                ploc_mine[rows, :] + prem_recv_s[rows, :])
        for c in range(NC):
            rows = slice(c * CH, (c + 1) * CH)
            ploc_rdmas[c].wait_recv()
            prem_rdmas[c][1].wait_recv()
            out_ref[pl.ds(other0 + c * CH, CH), :] = (
                ploc_recv[rows, :] + prem_recv_d[rows, :])

        rdma_x.wait_send()
        rdma_a.wait_send()
        for r in ploc_rdmas:
            r.wait_send()
        for rs, rd in prem_rdmas:
            rs.wait_send()
            rd.wait_send()

    return pl.pallas_call(
        body,
        out_shape=jax.ShapeDtypeStruct((T, D), jnp.bfloat16),
        in_specs=[
            pl.BlockSpec(memory_space=pltpu.VMEM),
            pl.BlockSpec(memory_space=pltpu.VMEM),
            pl.BlockSpec(memory_space=pltpu.VMEM),
            pl.BlockSpec(memory_space=pltpu.VMEM),
        ],
        out_specs=pl.BlockSpec(memory_space=pltpu.VMEM),
        scratch_shapes=[
            pltpu.VMEM((HH, D), jnp.bfloat16),
            pltpu.VMEM((HH, D), jnp.bfloat16),
            pltpu.VMEM((T, 1), jnp.int32),
            pltpu.VMEM((HH, D), jnp.bfloat16),
            pltpu.VMEM((HH, D), jnp.bfloat16),
            pltpu.VMEM((HH, D), jnp.bfloat16),
            pltpu.VMEM((HH, D), jnp.bfloat16),
            pltpu.VMEM((HH, D), jnp.bfloat16),
            pltpu.SemaphoreType.DMA((NSEM,)),
            pltpu.SemaphoreType.DMA((NSEM,)),
        ],
        compiler_params=pltpu.CompilerParams(collective_id=0),
    )(x, assign2d, w1b_in, w2b_in)
